# baseline (device time: 366473 ns/iter reference)
import jax
import jax.numpy as jnp
from jax import lax
from jax.experimental import pallas as pl
from jax.experimental.pallas import tpu as pltpu

T = 1024
D = 2048
VH = 16384
NCHUNK = 32
CW = VH // NCHUNK
NQ = 4
NM = NCHUNK // NQ
F32 = jnp.float32


def _fused(x, W):
    def body(x_ref, w_ref, out_ref, recv_ref, logits_ref, s_acc, s_other,
             out_stage, in_stage,
             z_send, z_recv, xd_send, xd_recv, yd_send, yd_recv,
             yt_send, yt_recv, xt_send, xt_recv,
             stat_send, stat_recv, in_copy, out_copy):
        j = pl.program_id(0)
        my_x = lax.axis_index("x")
        my_y = lax.axis_index("y")
        my_z = lax.axis_index("z")
        zpeer = (my_x, my_y, 1 - my_z)
        xpeer = (1 - my_x, my_y, my_z)
        ypeer = (my_x, 1 - my_y, my_z)

        q_me = my_x + 2 * my_y
        q_xp = (1 - my_x) + 2 * my_y
        q_yp = my_x + 2 * (1 - my_y)
        q_d = (1 - my_x) + 2 * (1 - my_y)

        def cols(ref, h):
            return ref.at[:, pl.ds(h * CW, CW)]

        def rdma(src, h, send_sem, recv_sem, peer):
            return pltpu.make_async_remote_copy(
                src_ref=src,
                dst_ref=cols(recv_ref, h),
                send_sem=send_sem,
                recv_sem=recv_sem,
                device_id=peer,
                device_id_type=pl.DeviceIdType.MESH,
            )

        def z_rdma(m):
            h = NQ * m + q_me
            return rdma(cols(logits_ref, h), h, z_send.at[m], z_recv.at[m],
                        zpeer)

        def xd_out(m):
            h = NQ * m + q_me
            return rdma(cols(recv_ref, h), h, xd_send.at[m], xd_recv.at[m],
                        xpeer)

        def yd_out(m):
            h = NQ * m + q_me
            return rdma(cols(recv_ref, h), h, yd_send.at[m], yd_recv.at[m],
                        ypeer)

        def xd_in(m):
            h = NQ * m + q_xp
            return rdma(cols(recv_ref, h), h, xd_send.at[m], xd_recv.at[m],
                        xpeer)

        def yd_in(m):
            h = NQ * m + q_yp
            return rdma(cols(recv_ref, h), h, yd_send.at[m], yd_recv.at[m],
                        ypeer)

        def yt_out(m):
            h = NQ * m + q_xp
            return rdma(cols(recv_ref, h), h, yt_send.at[m], yt_recv.at[m],
                        ypeer)

        def yt_in(m):
            h = NQ * m + q_d
            return rdma(cols(recv_ref, h), h, yt_send.at[m], yt_recv.at[m],
                        ypeer)

        def xt_out(m):
            h = NQ * m + q_yp
            return rdma(cols(recv_ref, h), h, xt_send.at[m - 4],
                        xt_recv.at[m - 4], xpeer)

        def xt_in(m):
            h = NQ * m + q_d
            return rdma(cols(recv_ref, h), h, xt_send.at[m - 4],
                        xt_recv.at[m - 4], xpeer)

        def stat_rdma():
            return pltpu.make_async_remote_copy(
                src_ref=s_acc,
                dst_ref=s_other,
                send_sem=stat_send,
                recv_sem=stat_recv,
                device_id=zpeer,
                device_id_type=pl.DeviceIdType.MESH,
            )

        @pl.when(j == 0)
        def _():
            barrier = pltpu.get_barrier_semaphore()
            for peer in (zpeer, xpeer, ypeer):
                pl.semaphore_signal(
                    barrier, inc=1, device_id=peer,
                    device_id_type=pl.DeviceIdType.MESH,
                )
            pl.semaphore_wait(barrier, 3)

        xb = x_ref[...].astype(jnp.bfloat16)
        wb = w_ref[...].astype(jnp.bfloat16)
        acc = jnp.dot(xb, wb, preferred_element_type=F32)
        logits_ref[:, pl.ds(j * CW, CW)] = acc.astype(jnp.bfloat16)
        ex_sum = jnp.sum(jnp.exp(acc), axis=1, keepdims=True)

        @pl.when(j == 0)
        def _():
            s_acc[...] = ex_sum

        @pl.when(j > 0)
        def _():
            s_acc[...] = s_acc[...] + ex_sum

        @pl.when(j % NQ == q_me)
        def _():
            z_rdma(j // NQ).start()

        @pl.when((j >= 4) & (j % NQ == 0))
        def _():
            m = j // NQ - 1
            z_rdma(m).wait_recv()
            xd_out(m).start()
            yd_out(m).start()

        @pl.when((j >= 20) & (j <= 29) & ((j - 20) % 3 == 0))
        def _():
            m = (j - 20) // 3
            xd_in(m).wait_recv()
            yt_out(m).start()

        @pl.when(j == NCHUNK - 1)
        def _():
            stat_rdma().start()

            z_rdma(NM - 1).wait_recv()
            xd_out(NM - 1).start()
            yd_out(NM - 1).start()

            for m in range(4, NM):
                yd_in(m).wait_recv()
                xt_out(m).start()

            stat_rdma().wait_recv()
            inv = 1.0 / (s_acc[...] + s_other[...])

            def my_chunk(c, _):
                slot = c % 2

                @pl.when(c >= 2)
                def _():
                    pltpu.make_async_copy(
                        out_stage.at[slot],
                        out_ref.at[:, pl.ds(my_z * VH + c * CW, CW)],
                        out_copy.at[slot],
                    ).wait()

                v = jnp.exp(
                    logits_ref[:, pl.ds(c * CW, CW)].astype(F32)
                ) * inv
                out_stage[slot] = v
                pltpu.make_async_copy(
                    out_stage.at[slot],
                    out_ref.at[:, pl.ds(my_z * VH + c * CW, CW)],
                    out_copy.at[slot],
                ).start()
                return 0

            lax.fori_loop(0, NCHUNK, my_chunk, 0)

            items = (
                [(NQ * m + q_me, None) for m in range(NM)]
                + [(NQ * m + q_xp, None) for m in range(4)]
                + [(NQ * m + q_yp, yd_in(m)) for m in range(4)]
                + [(NQ * m + q_xp, xd_in(m)) for m in range(4, NM)]
                + [(NQ * m + q_yp, None) for m in range(4, NM)]
                + [(NQ * m + q_d, yt_in(m)) for m in range(4)]
                + [(NQ * m + q_d, xt_in(m)) for m in range(4, NM)]
            )
            n = len(items)

            def copy_in(i):
                h, _ = items[i]
                return pltpu.make_async_copy(
                    cols(recv_ref, h), in_stage.at[i % 2], in_copy.at[i % 2]
                )

            h0, w0 = items[0]
            if w0 is not None:
                w0.wait_recv()
            copy_in(0).start()
            for i in range(n):
                h, _ = items[i]
                if i + 1 < n:
                    hn, wn = items[i + 1]
                    if wn is not None:
                        wn.wait_recv()
                    copy_in(i + 1).start()
                copy_in(i).wait()
                v = jnp.exp(in_stage[i % 2].astype(F32)) * inv
                slot = i % 2
                out_col = (1 - my_z) * VH + h * CW
                pltpu.make_async_copy(
                    out_stage.at[slot],
                    out_ref.at[:, pl.ds(out_col, CW)],
                    out_copy.at[slot],
                ).wait()
                out_stage[slot] = v
                pltpu.make_async_copy(
                    out_stage.at[slot],
                    out_ref.at[:, pl.ds(out_col, CW)],
                    out_copy.at[slot],
                ).start()

            for slot in range(2):
                pltpu.make_async_copy(
                    out_stage.at[slot],
                    out_ref.at[:, pl.ds(slot * CW, CW)],
                    out_copy.at[slot],
                ).wait()
            for m in range(NM):
                z_rdma(m).wait_send()
                xd_out(m).wait_send()
                yd_out(m).wait_send()
            for m in range(4):
                yt_out(m).wait_send()
            for m in range(4, NM):
                xt_out(m).wait_send()
            stat_rdma().wait_send()

    return pl.pallas_call(
        body,
        grid=(NCHUNK,),
        in_specs=[
            pl.BlockSpec((T, D), lambda j: (0, 0)),
            pl.BlockSpec((D, CW), lambda j: (0, j)),
        ],
        out_specs=[
            pl.BlockSpec(memory_space=pl.ANY),
            pl.BlockSpec(memory_space=pl.ANY),
        ],
        out_shape=[
            jax.ShapeDtypeStruct((T, 2 * VH), F32),
            jax.ShapeDtypeStruct((T, VH), jnp.bfloat16),
        ],
        scratch_shapes=[
            pltpu.VMEM((T, VH), jnp.bfloat16),
            pltpu.VMEM((T, 1), F32),
            pltpu.VMEM((T, 1), F32),
            pltpu.VMEM((2, T, CW), F32),
            pltpu.VMEM((2, T, CW), jnp.bfloat16),
            pltpu.SemaphoreType.DMA((NM,)),
            pltpu.SemaphoreType.DMA((NM,)),
            pltpu.SemaphoreType.DMA((NM,)),
            pltpu.SemaphoreType.DMA((NM,)),
            pltpu.SemaphoreType.DMA((NM,)),
            pltpu.SemaphoreType.DMA((NM,)),
            pltpu.SemaphoreType.DMA((4,)),
            pltpu.SemaphoreType.DMA((4,)),
            pltpu.SemaphoreType.DMA((4,)),
            pltpu.SemaphoreType.DMA((4,)),
            pltpu.SemaphoreType.DMA,
            pltpu.SemaphoreType.DMA,
            pltpu.SemaphoreType.DMA((2,)),
            pltpu.SemaphoreType.DMA((2,)),
        ],
        compiler_params=pltpu.CompilerParams(
            collective_id=0, vmem_limit_bytes=100 * 1024 * 1024
        ),
    )(x, W)


def kernel(x, W):
    out, _ = _fused(x, W)
    return out


# device time: 362639 ns/iter; 1.0106x vs baseline; 1.0106x over previous
import jax
import jax.numpy as jnp
from jax import lax
from jax.experimental import pallas as pl
from jax.experimental.pallas import tpu as pltpu

T = 1024
D = 2048
VH = 16384
NCHUNK = 32
CW = VH // NCHUNK
NQ = 4
NM = NCHUNK // NQ
F32 = jnp.float32


def _fused(x, W):
    def body(x_ref, w_ref, out_ref, recv_ref, logits_ref, s_acc, s_other,
             out_stage, in_stage,
             z_send, z_recv, xd_send, xd_recv, yd_send, yd_recv,
             yt_send, yt_recv, xt_send, xt_recv,
             stat_send, stat_recv, in_copy, out_copy):
        j = pl.program_id(0)
        my_x = lax.axis_index("x")
        my_y = lax.axis_index("y")
        my_z = lax.axis_index("z")
        zpeer = (my_x, my_y, 1 - my_z)
        xpeer = (1 - my_x, my_y, my_z)
        ypeer = (my_x, 1 - my_y, my_z)

        q_me = my_x + 2 * my_y
        q_xp = (1 - my_x) + 2 * my_y
        q_yp = my_x + 2 * (1 - my_y)
        q_d = (1 - my_x) + 2 * (1 - my_y)

        def cols(ref, h):
            return ref.at[:, pl.ds(h * CW, CW)]

        def rdma(src, h, send_sem, recv_sem, peer):
            return pltpu.make_async_remote_copy(
                src_ref=src,
                dst_ref=cols(recv_ref, h),
                send_sem=send_sem,
                recv_sem=recv_sem,
                device_id=peer,
                device_id_type=pl.DeviceIdType.MESH,
            )

        def z_rdma(m):
            h = NQ * m + q_me
            return rdma(cols(logits_ref, h), h, z_send.at[m], z_recv.at[m],
                        zpeer)

        def xd_out(m):
            h = NQ * m + q_me
            return rdma(cols(recv_ref, h), h, xd_send.at[m], xd_recv.at[m],
                        xpeer)

        def yd_out(m):
            h = NQ * m + q_me
            return rdma(cols(recv_ref, h), h, yd_send.at[m], yd_recv.at[m],
                        ypeer)

        def xd_in(m):
            h = NQ * m + q_xp
            return rdma(cols(recv_ref, h), h, xd_send.at[m], xd_recv.at[m],
                        xpeer)

        def yd_in(m):
            h = NQ * m + q_yp
            return rdma(cols(recv_ref, h), h, yd_send.at[m], yd_recv.at[m],
                        ypeer)

        def yt_out(m):
            h = NQ * m + q_xp
            return rdma(cols(recv_ref, h), h, yt_send.at[m], yt_recv.at[m],
                        ypeer)

        def yt_in(m):
            h = NQ * m + q_d
            return rdma(cols(recv_ref, h), h, yt_send.at[m], yt_recv.at[m],
                        ypeer)

        def xt_out(m):
            h = NQ * m + q_yp
            return rdma(cols(recv_ref, h), h, xt_send.at[m - 4],
                        xt_recv.at[m - 4], xpeer)

        def xt_in(m):
            h = NQ * m + q_d
            return rdma(cols(recv_ref, h), h, xt_send.at[m - 4],
                        xt_recv.at[m - 4], xpeer)

        def stat_rdma():
            return pltpu.make_async_remote_copy(
                src_ref=s_acc,
                dst_ref=s_other,
                send_sem=stat_send,
                recv_sem=stat_recv,
                device_id=zpeer,
                device_id_type=pl.DeviceIdType.MESH,
            )

        @pl.when(j == 0)
        def _():
            barrier = pltpu.get_barrier_semaphore()
            for peer in (zpeer, xpeer, ypeer):
                pl.semaphore_signal(
                    barrier, inc=1, device_id=peer,
                    device_id_type=pl.DeviceIdType.MESH,
                )
            pl.semaphore_wait(barrier, 3)

        xb = x_ref[...].astype(jnp.bfloat16)
        wb = w_ref[...].astype(jnp.bfloat16)
        acc = jnp.dot(xb, wb, preferred_element_type=F32)
        eacc = jnp.exp(acc)
        logits_ref[:, pl.ds(j * CW, CW)] = eacc.astype(jnp.bfloat16)
        ex_sum = jnp.sum(eacc, axis=1, keepdims=True)

        @pl.when(j == 0)
        def _():
            s_acc[...] = ex_sum

        @pl.when(j > 0)
        def _():
            s_acc[...] = s_acc[...] + ex_sum

        @pl.when(j % NQ == q_me)
        def _():
            z_rdma(j // NQ).start()

        @pl.when((j >= 4) & (j % NQ == 0))
        def _():
            m = j // NQ - 1
            z_rdma(m).wait_recv()
            xd_out(m).start()
            yd_out(m).start()

        @pl.when((j >= 20) & (j <= 29) & ((j - 20) % 3 == 0))
        def _():
            m = (j - 20) // 3
            xd_in(m).wait_recv()
            yt_out(m).start()

        @pl.when(j == NCHUNK - 1)
        def _():
            stat_rdma().start()

            z_rdma(NM - 1).wait_recv()
            xd_out(NM - 1).start()
            yd_out(NM - 1).start()

            for m in range(4, NM):
                yd_in(m).wait_recv()
                xt_out(m).start()

            stat_rdma().wait_recv()
            inv = 1.0 / (s_acc[...] + s_other[...])

            def my_chunk(c, _):
                slot = c % 2

                @pl.when(c >= 2)
                def _():
                    pltpu.make_async_copy(
                        out_stage.at[slot],
                        out_ref.at[:, pl.ds(my_z * VH + c * CW, CW)],
                        out_copy.at[slot],
                    ).wait()

                v = logits_ref[:, pl.ds(c * CW, CW)].astype(F32) * inv
                out_stage[slot] = v
                pltpu.make_async_copy(
                    out_stage.at[slot],
                    out_ref.at[:, pl.ds(my_z * VH + c * CW, CW)],
                    out_copy.at[slot],
                ).start()
                return 0

            lax.fori_loop(0, NCHUNK, my_chunk, 0)

            items = (
                [(NQ * m + q_me, None) for m in range(NM)]
                + [(NQ * m + q_xp, None) for m in range(4)]
                + [(NQ * m + q_yp, yd_in(m)) for m in range(4)]
                + [(NQ * m + q_xp, xd_in(m)) for m in range(4, NM)]
                + [(NQ * m + q_yp, None) for m in range(4, NM)]
                + [(NQ * m + q_d, yt_in(m)) for m in range(4)]
                + [(NQ * m + q_d, xt_in(m)) for m in range(4, NM)]
            )
            n = len(items)

            def copy_in(i):
                h, _ = items[i]
                return pltpu.make_async_copy(
                    cols(recv_ref, h), in_stage.at[i % 2], in_copy.at[i % 2]
                )

            h0, w0 = items[0]
            if w0 is not None:
                w0.wait_recv()
            copy_in(0).start()
            for i in range(n):
                h, _ = items[i]
                if i + 1 < n:
                    hn, wn = items[i + 1]
                    if wn is not None:
                        wn.wait_recv()
                    copy_in(i + 1).start()
                copy_in(i).wait()
                v = in_stage[i % 2].astype(F32) * inv
                slot = i % 2
                out_col = (1 - my_z) * VH + h * CW
                pltpu.make_async_copy(
                    out_stage.at[slot],
                    out_ref.at[:, pl.ds(out_col, CW)],
                    out_copy.at[slot],
                ).wait()
                out_stage[slot] = v
                pltpu.make_async_copy(
                    out_stage.at[slot],
                    out_ref.at[:, pl.ds(out_col, CW)],
                    out_copy.at[slot],
                ).start()

            for slot in range(2):
                pltpu.make_async_copy(
                    out_stage.at[slot],
                    out_ref.at[:, pl.ds(slot * CW, CW)],
                    out_copy.at[slot],
                ).wait()
            for m in range(NM):
                z_rdma(m).wait_send()
                xd_out(m).wait_send()
                yd_out(m).wait_send()
            for m in range(4):
                yt_out(m).wait_send()
            for m in range(4, NM):
                xt_out(m).wait_send()
            stat_rdma().wait_send()

    return pl.pallas_call(
        body,
        grid=(NCHUNK,),
        in_specs=[
            pl.BlockSpec((T, D), lambda j: (0, 0)),
            pl.BlockSpec((D, CW), lambda j: (0, j)),
        ],
        out_specs=[
            pl.BlockSpec(memory_space=pl.ANY),
            pl.BlockSpec(memory_space=pl.ANY),
        ],
        out_shape=[
            jax.ShapeDtypeStruct((T, 2 * VH), F32),
            jax.ShapeDtypeStruct((T, VH), jnp.bfloat16),
        ],
        scratch_shapes=[
            pltpu.VMEM((T, VH), jnp.bfloat16),
            pltpu.VMEM((T, 1), F32),
            pltpu.VMEM((T, 1), F32),
            pltpu.VMEM((2, T, CW), F32),
            pltpu.VMEM((2, T, CW), jnp.bfloat16),
            pltpu.SemaphoreType.DMA((NM,)),
            pltpu.SemaphoreType.DMA((NM,)),
            pltpu.SemaphoreType.DMA((NM,)),
            pltpu.SemaphoreType.DMA((NM,)),
            pltpu.SemaphoreType.DMA((NM,)),
            pltpu.SemaphoreType.DMA((NM,)),
            pltpu.SemaphoreType.DMA((4,)),
            pltpu.SemaphoreType.DMA((4,)),
            pltpu.SemaphoreType.DMA((4,)),
            pltpu.SemaphoreType.DMA((4,)),
            pltpu.SemaphoreType.DMA,
            pltpu.SemaphoreType.DMA,
            pltpu.SemaphoreType.DMA((2,)),
            pltpu.SemaphoreType.DMA((2,)),
        ],
        compiler_params=pltpu.CompilerParams(
            collective_id=0, vmem_limit_bytes=100 * 1024 * 1024
        ),
    )(x, W)


def kernel(x, W):
    out, _ = _fused(x, W)
    return out


# device time: 361749 ns/iter; 1.0131x vs baseline; 1.0025x over previous
import jax
import jax.numpy as jnp
from jax import lax
from jax.experimental import pallas as pl
from jax.experimental.pallas import tpu as pltpu

T = 1024
D = 2048
VH = 16384
NCHUNK = 32
CW = VH // NCHUNK
NQ = 4
NM = NCHUNK // NQ
F32 = jnp.float32
SKELETON = False


def _fused(x, W):
    def body(x_ref, w_ref, out_ref, recv_ref, logits_ref, s_acc, s_other,
             out_stage, in_stage,
             z_send, z_recv, xd_send, xd_recv, yd_send, yd_recv,
             yt_send, yt_recv, xt_send, xt_recv,
             stat_send, stat_recv, in_copy, out_copy):
        j = pl.program_id(0)
        my_x = lax.axis_index("x")
        my_y = lax.axis_index("y")
        my_z = lax.axis_index("z")
        zpeer = (my_x, my_y, 1 - my_z)
        xpeer = (1 - my_x, my_y, my_z)
        ypeer = (my_x, 1 - my_y, my_z)

        q_me = my_x + 2 * my_y
        q_xp = (1 - my_x) + 2 * my_y
        q_yp = my_x + 2 * (1 - my_y)
        q_d = (1 - my_x) + 2 * (1 - my_y)

        def cols(ref, h):
            return ref.at[:, pl.ds(h * CW, CW)]

        def rdma(src, h, send_sem, recv_sem, peer):
            return pltpu.make_async_remote_copy(
                src_ref=src,
                dst_ref=cols(recv_ref, h),
                send_sem=send_sem,
                recv_sem=recv_sem,
                device_id=peer,
                device_id_type=pl.DeviceIdType.MESH,
            )

        def z_rdma(m):
            h = NQ * m + q_me
            return rdma(cols(logits_ref, h), h, z_send.at[m], z_recv.at[m],
                        zpeer)

        def xd_out(m):
            h = NQ * m + q_me
            return rdma(cols(recv_ref, h), h, xd_send.at[m], xd_recv.at[m],
                        xpeer)

        def yd_out(m):
            h = NQ * m + q_me
            return rdma(cols(recv_ref, h), h, yd_send.at[m], yd_recv.at[m],
                        ypeer)

        def xd_in(m):
            h = NQ * m + q_xp
            return rdma(cols(recv_ref, h), h, xd_send.at[m], xd_recv.at[m],
                        xpeer)

        def yd_in(m):
            h = NQ * m + q_yp
            return rdma(cols(recv_ref, h), h, yd_send.at[m], yd_recv.at[m],
                        ypeer)

        def yt_out(m):
            h = NQ * m + q_xp
            return rdma(cols(recv_ref, h), h, yt_send.at[m], yt_recv.at[m],
                        ypeer)

        def yt_in(m):
            h = NQ * m + q_d
            return rdma(cols(recv_ref, h), h, yt_send.at[m], yt_recv.at[m],
                        ypeer)

        def xt_out(m):
            h = NQ * m + q_yp
            return rdma(cols(recv_ref, h), h, xt_send.at[m - 4],
                        xt_recv.at[m - 4], xpeer)

        def xt_in(m):
            h = NQ * m + q_d
            return rdma(cols(recv_ref, h), h, xt_send.at[m - 4],
                        xt_recv.at[m - 4], xpeer)

        def stat_rdma():
            return pltpu.make_async_remote_copy(
                src_ref=s_acc,
                dst_ref=s_other,
                send_sem=stat_send,
                recv_sem=stat_recv,
                device_id=zpeer,
                device_id_type=pl.DeviceIdType.MESH,
            )

        @pl.when(j == 0)
        def _():
            barrier = pltpu.get_barrier_semaphore()
            for peer in (zpeer, xpeer, ypeer):
                pl.semaphore_signal(
                    barrier, inc=1, device_id=peer,
                    device_id_type=pl.DeviceIdType.MESH,
                )
            pl.semaphore_wait(barrier, 3)

        xb = x_ref[...].astype(jnp.bfloat16)
        wb = w_ref[...].astype(jnp.bfloat16)
        acc = jnp.dot(xb, wb, preferred_element_type=F32)
        logits_ref[:, pl.ds(j * CW, CW)] = acc.astype(jnp.bfloat16)

        @pl.when(j % NQ == q_me)
        def _():
            z_rdma(j // NQ).start()

        @pl.when((j >= 4) & (j % NQ == 0))
        def _():
            m = j // NQ - 1
            z_rdma(m).wait_recv()
            xd_out(m).start()
            yd_out(m).start()

        @pl.when((j >= 20) & (j <= 29) & ((j - 20) % 3 == 0))
        def _():
            m = (j - 20) // 3
            xd_in(m).wait_recv()
            yt_out(m).start()

        @pl.when(j == NCHUNK - 1)
        def _():
            z_rdma(NM - 1).wait_recv()
            xd_out(NM - 1).start()
            yd_out(NM - 1).start()

            def stat_chunk(c, s):
                return s + jnp.sum(
                    jnp.exp(logits_ref[:, pl.ds(c * CW, CW)].astype(F32)),
                    axis=1,
                    keepdims=True,
                )

            s_acc[...] = lax.fori_loop(
                0, NCHUNK, stat_chunk, jnp.zeros((T, 1), F32)
            )
            stat_rdma().start()

            for m in range(4, NM):
                yd_in(m).wait_recv()
                xt_out(m).start()

            stat_rdma().wait_recv()
            inv = 1.0 / (s_acc[...] + s_other[...])

            def my_chunk(c, _):
                slot = c % 2

                @pl.when(c >= 2)
                def _():
                    pltpu.make_async_copy(
                        out_stage.at[slot],
                        out_ref.at[:, pl.ds(my_z * VH + c * CW, CW)],
                        out_copy.at[slot],
                    ).wait()

                v = jnp.exp(
                    logits_ref[:, pl.ds(c * CW, CW)].astype(F32)
                ) * inv
                out_stage[slot] = v
                pltpu.make_async_copy(
                    out_stage.at[slot],
                    out_ref.at[:, pl.ds(my_z * VH + c * CW, CW)],
                    out_copy.at[slot],
                ).start()
                return 0

            if not SKELETON:
                lax.fori_loop(0, NCHUNK, my_chunk, 0)

            items = (
                [(NQ * m + q_me, None) for m in range(NM)]
                + [(NQ * m + q_xp, None) for m in range(4)]
                + [(NQ * m + q_yp, yd_in(m)) for m in range(4)]
                + [(NQ * m + q_xp, xd_in(m)) for m in range(4, NM)]
                + [(NQ * m + q_yp, None) for m in range(4, NM)]
                + [(NQ * m + q_d, yt_in(m)) for m in range(4)]
                + [(NQ * m + q_d, xt_in(m)) for m in range(4, NM)]
            )
            n = len(items)

            def copy_in(i):
                h, _ = items[i]
                return pltpu.make_async_copy(
                    cols(recv_ref, h), in_stage.at[i % 2], in_copy.at[i % 2]
                )

            if SKELETON:
                for _, wd in items:
                    if wd is not None:
                        wd.wait_recv()
            else:
                h0, w0 = items[0]
                if w0 is not None:
                    w0.wait_recv()
                copy_in(0).start()
                for i in range(n):
                    h, _ = items[i]
                    if i + 1 < n:
                        hn, wn = items[i + 1]
                        if wn is not None:
                            wn.wait_recv()
                        copy_in(i + 1).start()
                    copy_in(i).wait()
                    v = jnp.exp(in_stage[i % 2].astype(F32)) * inv
                    slot = i % 2
                    out_col = (1 - my_z) * VH + h * CW
                    pltpu.make_async_copy(
                        out_stage.at[slot],
                        out_ref.at[:, pl.ds(out_col, CW)],
                        out_copy.at[slot],
                    ).wait()
                    out_stage[slot] = v
                    pltpu.make_async_copy(
                        out_stage.at[slot],
                        out_ref.at[:, pl.ds(out_col, CW)],
                        out_copy.at[slot],
                    ).start()

                for slot in range(2):
                    pltpu.make_async_copy(
                        out_stage.at[slot],
                        out_ref.at[:, pl.ds(slot * CW, CW)],
                        out_copy.at[slot],
                    ).wait()
            for m in range(NM):
                z_rdma(m).wait_send()
                xd_out(m).wait_send()
                yd_out(m).wait_send()
            for m in range(4):
                yt_out(m).wait_send()
            for m in range(4, NM):
                xt_out(m).wait_send()
            stat_rdma().wait_send()

    return pl.pallas_call(
        body,
        grid=(NCHUNK,),
        in_specs=[
            pl.BlockSpec((T, D), lambda j: (0, 0)),
            pl.BlockSpec((D, CW), lambda j: (0, j)),
        ],
        out_specs=[
            pl.BlockSpec(memory_space=pl.ANY),
            pl.BlockSpec(memory_space=pl.ANY),
        ],
        out_shape=[
            jax.ShapeDtypeStruct((T, 2 * VH), F32),
            jax.ShapeDtypeStruct((T, VH), jnp.bfloat16),
        ],
        scratch_shapes=[
            pltpu.VMEM((T, VH), jnp.bfloat16),
            pltpu.VMEM((T, 1), F32),
            pltpu.VMEM((T, 1), F32),
            pltpu.VMEM((2, T, CW), F32),
            pltpu.VMEM((2, T, CW), jnp.bfloat16),
            pltpu.SemaphoreType.DMA((NM,)),
            pltpu.SemaphoreType.DMA((NM,)),
            pltpu.SemaphoreType.DMA((NM,)),
            pltpu.SemaphoreType.DMA((NM,)),
            pltpu.SemaphoreType.DMA((NM,)),
            pltpu.SemaphoreType.DMA((NM,)),
            pltpu.SemaphoreType.DMA((4,)),
            pltpu.SemaphoreType.DMA((4,)),
            pltpu.SemaphoreType.DMA((4,)),
            pltpu.SemaphoreType.DMA((4,)),
            pltpu.SemaphoreType.DMA,
            pltpu.SemaphoreType.DMA,
            pltpu.SemaphoreType.DMA((2,)),
            pltpu.SemaphoreType.DMA((2,)),
        ],
        compiler_params=pltpu.CompilerParams(
            collective_id=0, vmem_limit_bytes=100 * 1024 * 1024
        ),
    )(x, W)


def kernel(x, W):
    out, _ = _fused(x, W)
    return out
